# baseline (speedup 1.0000x reference)
import jax
import jax.numpy as jnp
from jax import lax
from jax.experimental import pallas as pl
from jax.experimental.pallas import tpu as pltpu
from jax.experimental.pallas import tpu_sc as plsc

B = 4096
NA = 10
NE = 20
NC = 2
NS = 16
NW = NC * NS
NT = B // 128
KS = 128 // 16


def _vsqrt(x):
    i = plsc.bitcast(x, jnp.int32)
    i = jnp.int32(0x5F3759DF) - (i >> 1)
    y = plsc.bitcast(i, jnp.float32)
    for _ in range(2):
        y = y * (1.5 - 0.5 * x * y * y)
    return x * y


def _sc_body(mov_hbm, loc_hbm, goal_hbm, out_hbm, mov_v, loc_v, goal_v, res_v, sem):
    wid = lax.axis_index("s") * NC + lax.axis_index("c")
    cps = (
        [
            pltpu.async_copy(
                mov_hbm.at[pl.ds(0, NA), wid, c], mov_v.at[pl.ds(c * NA, NA)], sem
            )
            for c in range(2)
        ]
        + [
            pltpu.async_copy(
                loc_hbm.at[pl.ds(0, NA), wid, c], loc_v.at[pl.ds(c * NA, NA)], sem
            )
            for c in range(2)
        ]
        + [
            pltpu.async_copy(
                goal_hbm.at[c, :, wid], goal_v.at[pl.ds(c * NA, NA)], sem
            )
            for c in range(3)
        ]
    )
    for c in cps:
        c.wait()

    lane = lax.broadcasted_iota(jnp.int32, (16,), 0)

    def phys_step(t, acc):
        j = t // KS
        col = (t % KS) * 16 + lane
        gx = plsc.load_gather(goal_v, [j + lane * 0, col])
        gy = plsc.load_gather(goal_v, [j + NA + lane * 0, col])
        g = plsc.load_gather(goal_v, [j + 2 * NA + lane * 0, col]).astype(jnp.int32)
        lx = plsc.load_gather(loc_v, [g, col]) + plsc.load_gather(mov_v, [g, col])
        ly = plsc.load_gather(loc_v, [g + NA, col]) + plsc.load_gather(
            mov_v, [g + NA, col]
        )
        dx = lx - gx
        dy = ly - gy
        return acc + _vsqrt(dx * dx + dy * dy)

    res_v[...] = lax.fori_loop(0, KS * NA, phys_step, jnp.zeros((16,), jnp.float32))
    pltpu.sync_copy(res_v, out_hbm.at[wid])


def _tc_body(x_ref, o_ref):
    x = x_ref[...]
    mx = x[:, :128]
    my = x[:, 128:]
    n = jnp.sqrt(mx * mx + my * my)
    o_ref[...] = jnp.sum(n).reshape(1, 1)


@jax.jit
def _cost(mov4, loc4, goal4, mov2):
    mesh = plsc.VectorSubcoreMesh(core_axis_name="c", subcore_axis_name="s")
    f = pl.kernel(
        _sc_body,
        out_type=jax.ShapeDtypeStruct((NW, 16), jnp.float32),
        mesh=mesh,
        scratch_types=[
            pltpu.VMEM((2 * NA, 128), jnp.float32),
            pltpu.VMEM((2 * NA, 128), jnp.float32),
            pltpu.VMEM((3 * NA, 128), jnp.float32),
            pltpu.VMEM((16,), jnp.float32),
            pltpu.SemaphoreType.DMA,
        ],
        compiler_params=pltpu.CompilerParams(needs_layout_passes=False),
    )
    phys = f(mov4, loc4, goal4)
    move = pl.pallas_call(
        _tc_body,
        out_shape=jax.ShapeDtypeStruct((1, 1), jnp.float32),
    )(mov2)
    return 2.0 * jnp.sum(phys) + move[0, 0]


def kernel(movements, goal_predictions, utterances, locations, goals):
    mov4 = movements.transpose(1, 2, 0).reshape(NE, 2, NT, 128).transpose(0, 2, 1, 3)
    loc4 = locations.transpose(1, 2, 0).reshape(NE, 2, NT, 128).transpose(0, 2, 1, 3)
    mov2 = mov4.reshape(NE * NT, 256)
    goal4 = goals.transpose(2, 1, 0).reshape(3, NA, NT, 128)
    return _cost(mov4, loc4, goal4, mov2)


# speedup vs baseline: 1.0166x; 1.0166x over previous
import jax
import jax.numpy as jnp
from jax import lax
from jax.experimental import pallas as pl
from jax.experimental.pallas import tpu as pltpu
from jax.experimental.pallas import tpu_sc as plsc

B = 4096
NA = 10
NE = 20
NC = 2
NS = 16
NW = NC * NS
NT = B // 128
KS = 128 // 16


def _vsqrt(x):
    i = plsc.bitcast(x, jnp.int32)
    i = jnp.int32(0x5F3759DF) - (i >> 1)
    y = plsc.bitcast(i, jnp.float32)
    for _ in range(2):
        y = y * (1.5 - 0.5 * x * y * y)
    return x * y


def _sc_body(mov_hbm, loc_hbm, goal_hbm, out_hbm, mov_v, loc_v, goal_v, res_v, sem):
    wid = lax.axis_index("s") * NC + lax.axis_index("c")
    cps = (
        [
            pltpu.async_copy(
                mov_hbm.at[pl.ds(0, NA), wid, c], mov_v.at[pl.ds(c * NA, NA)], sem
            )
            for c in range(2)
        ]
        + [
            pltpu.async_copy(
                loc_hbm.at[pl.ds(0, NA), wid, c], loc_v.at[pl.ds(c * NA, NA)], sem
            )
            for c in range(2)
        ]
        + [
            pltpu.async_copy(
                goal_hbm.at[c, :, wid], goal_v.at[pl.ds(c * NA, NA)], sem
            )
            for c in range(3)
        ]
    )
    for c in cps:
        c.wait()

    lane = lax.broadcasted_iota(jnp.int32, (16,), 0)

    def phys_step(t, acc):
        j = t // KS
        col = (t % KS) * 16 + lane
        gx = plsc.load_gather(goal_v, [j + lane * 0, col])
        gy = plsc.load_gather(goal_v, [j + NA + lane * 0, col])
        g = plsc.load_gather(goal_v, [j + 2 * NA + lane * 0, col]).astype(jnp.int32)
        lx = plsc.load_gather(loc_v, [g, col]) + plsc.load_gather(mov_v, [g, col])
        ly = plsc.load_gather(loc_v, [g + NA, col]) + plsc.load_gather(
            mov_v, [g + NA, col]
        )
        dx = lx - gx
        dy = ly - gy
        return acc + _vsqrt(dx * dx + dy * dy)

    res_v[...] = lax.fori_loop(0, KS * NA, phys_step, jnp.zeros((16,), jnp.float32))
    pltpu.sync_copy(res_v, out_hbm.at[wid])


def _tc_body(x_ref, o_ref):
    mx = x_ref[:, :, 0, :]
    my = x_ref[:, :, 1, :]
    n = jnp.sqrt(mx * mx + my * my)
    o_ref[...] = jnp.sum(n).reshape(1, 1)


@jax.jit
def _cost(mov4, loc4, goal4):
    mesh = plsc.VectorSubcoreMesh(core_axis_name="c", subcore_axis_name="s")
    f = pl.kernel(
        _sc_body,
        out_type=jax.ShapeDtypeStruct((NW, 16), jnp.float32),
        mesh=mesh,
        scratch_types=[
            pltpu.VMEM((2 * NA, 128), jnp.float32),
            pltpu.VMEM((2 * NA, 128), jnp.float32),
            pltpu.VMEM((3 * NA, 128), jnp.float32),
            pltpu.VMEM((16,), jnp.float32),
            pltpu.SemaphoreType.DMA,
        ],
        compiler_params=pltpu.CompilerParams(needs_layout_passes=False),
    )
    phys = f(mov4, loc4, goal4)
    move = pl.pallas_call(
        _tc_body,
        out_shape=jax.ShapeDtypeStruct((1, 1), jnp.float32),
    )(mov4)
    return 2.0 * jnp.sum(phys) + move[0, 0]


def kernel(movements, goal_predictions, utterances, locations, goals):
    mov4 = movements.transpose(1, 2, 0).reshape(NE, 2, NT, 128).transpose(0, 2, 1, 3)
    loc4 = locations.transpose(1, 2, 0).reshape(NE, 2, NT, 128).transpose(0, 2, 1, 3)
    goal4 = goals.transpose(2, 1, 0).reshape(3, NA, NT, 128)
    return _cost(mov4, loc4, goal4)


# speedup vs baseline: 1.0525x; 1.0353x over previous
import jax
import jax.numpy as jnp
from jax import lax
from jax.experimental import pallas as pl
from jax.experimental.pallas import tpu as pltpu
from jax.experimental.pallas import tpu_sc as plsc

B = 4096
NA = 10
NE = 20
NC = 2
NS = 16
NW = NC * NS
NT = B // 128
KS = 128 // 16


def _vsqrt(x):
    i = plsc.bitcast(x, jnp.int32)
    i = jnp.int32(0x5F3759DF) - (i >> 1)
    y = plsc.bitcast(i, jnp.float32)
    for _ in range(2):
        y = y * (1.5 - 0.5 * x * y * y)
    return x * y


def _sc_body(
    mov_hbm, loc_hbm, goal_hbm, out_hbm, mov_v, loc_v, goal_v, res_v, sem_m, sem_lg
):
    wid = lax.axis_index("s") * NC + lax.axis_index("c")
    cps = [
        pltpu.async_copy(mov_hbm.at[:, wid, c], mov_v.at[pl.ds(c * NE, NE)], sem_m)
        for c in range(2)
    ]
    cps_lg = [
        pltpu.async_copy(loc_hbm.at[:, wid, c], loc_v.at[pl.ds(c * NE, NE)], sem_lg)
        for c in range(2)
    ] + [
        pltpu.async_copy(goal_hbm.at[:, pl.ds(wid * 128, 128)], goal_v, sem_lg)
    ]

    lane = lax.broadcasted_iota(jnp.int32, (16,), 0)

    def row(v):
        return jnp.full((16,), v, jnp.int32)

    for c in cps:
        c.wait()

    def move_step(t, acc):
        e = t // KS
        col = (t % KS) * 16 + lane
        mx = plsc.load_gather(mov_v, [e + lane * 0, col])
        my = plsc.load_gather(mov_v, [e + NE + lane * 0, col])
        return acc + _vsqrt(mx * mx + my * my)

    acc = lax.fori_loop(0, KS * NE, move_step, jnp.zeros((16,), jnp.float32))

    for c in cps_lg:
        c.wait()

    def phys_step(t, acc):
        j = t // KS
        col = (t % KS) * 16 + lane
        gx = plsc.load_gather(goal_v, [j + lane * 0, col])
        gy = plsc.load_gather(goal_v, [j + NA + lane * 0, col])
        g = plsc.load_gather(goal_v, [j + 2 * NA + lane * 0, col]).astype(jnp.int32)
        lx = plsc.load_gather(loc_v, [g, col]) + plsc.load_gather(mov_v, [g, col])
        ly = plsc.load_gather(loc_v, [g + NE, col]) + plsc.load_gather(
            mov_v, [g + NE, col]
        )
        dx = lx - gx
        dy = ly - gy
        return acc + 2.0 * _vsqrt(dx * dx + dy * dy)

    res_v[...] = lax.fori_loop(0, KS * NA, phys_step, acc)
    pltpu.sync_copy(res_v, out_hbm.at[wid])


@jax.jit
def _sc_cost(mov4, loc4, goal4):
    mesh = plsc.VectorSubcoreMesh(core_axis_name="c", subcore_axis_name="s")
    f = pl.kernel(
        _sc_body,
        out_type=jax.ShapeDtypeStruct((NW, 16), jnp.float32),
        mesh=mesh,
        scratch_types=[
            pltpu.VMEM((2 * NE, 128), jnp.float32),
            pltpu.VMEM((2 * NE, 128), jnp.float32),
            pltpu.VMEM((3 * NA, 128), jnp.float32),
            pltpu.VMEM((16,), jnp.float32),
            pltpu.SemaphoreType.DMA,
            pltpu.SemaphoreType.DMA,
        ],
        compiler_params=pltpu.CompilerParams(needs_layout_passes=False),
    )
    return jnp.sum(f(mov4, loc4, goal4))


def kernel(movements, goal_predictions, utterances, locations, goals):
    mov4 = movements.transpose(1, 2, 0).reshape(NE, 2, NT, 128).transpose(0, 2, 1, 3)
    loc4 = locations.transpose(1, 2, 0).reshape(NE, 2, NT, 128).transpose(0, 2, 1, 3)
    goal4 = goals.transpose(2, 1, 0).reshape(3 * NA, B)
    return _sc_cost(mov4, loc4, goal4)
